# Initial kernel scaffold; baseline (speedup 1.0000x reference)
import jax
import jax.numpy as jnp
from jax.experimental import pallas as pl


def kernel(x, edge_index, W1, b1, W2, b2):
    raise NotImplementedError("write your pallas kernel here")


# speedup vs baseline: 12.6618x; 12.6618x over previous
import functools

import jax
import jax.numpy as jnp
from jax import lax
from jax.experimental import pallas as pl
from jax.experimental.pallas import tpu as pltpu
from jax.experimental.pallas import tpu_sc as plsc

N = 10000
E = 320000
D_IN = 128
HIDDEN = 128
CLS = 40

NP = 10240
D2P = 128

NC, NS = 2, 16
NW = NC * NS
EPT = E // NW
CH = 80
ITERS = EPT // CH
RPS = NP // NS


@functools.cache
def _get_sc_degree():
    mesh = plsc.VectorSubcoreMesh(core_axis_name="c", subcore_axis_name="s",
                                  num_cores=NC, num_subcores=NS)
    return functools.partial(
        pl.kernel,
        out_type=jax.ShapeDtypeStruct((NC * NP,), jnp.float32),
        mesh=mesh,
        scratch_types=[
            pltpu.VMEM((CH,), jnp.int32),
            pltpu.VMEM((CH,), jnp.float32),
            pltpu.VMEM((RPS,), jnp.float32),
            pltpu.VMEM_SHARED((NP,), jnp.float32),
        ],
    )(_sc_degree_body)


def _sc_degree_body(dst_hbm, out_hbm, idx_v, ones_v, zbuf_v, acc_sh):
    c = lax.axis_index("c")
    s = lax.axis_index("s")
    wid = s * NC + c
    for i in range(RPS // 16):
        zbuf_v[pl.ds(i * 16, 16)] = jnp.zeros((16,), jnp.float32)
    for i in range(CH // 16):
        ones_v[pl.ds(i * 16, 16)] = jnp.ones((16,), jnp.float32)
    pltpu.sync_copy(zbuf_v, acc_sh.at[pl.ds(s * RPS, RPS)])
    plsc.subcore_barrier()

    def body(it, carry):
        base = wid * EPT + it * CH
        pltpu.sync_copy(dst_hbm.at[pl.ds(base, CH)], idx_v)
        pltpu.sync_copy(ones_v, acc_sh.at[idx_v], add=True)
        return carry

    lax.fori_loop(0, ITERS, body, 0)
    plsc.subcore_barrier()
    pltpu.sync_copy(acc_sh.at[pl.ds(s * RPS, RPS)],
                    out_hbm.at[pl.ds(c * NP + s * RPS, RPS)])


@functools.cache
def _make_sc_agg(D):
    mesh = plsc.VectorSubcoreMesh(core_axis_name="c", subcore_axis_name="s",
                                  num_cores=NC, num_subcores=NS)

    @functools.partial(
        pl.kernel,
        out_type=jax.ShapeDtypeStruct((NC * NP, D), jnp.float32),
        mesh=mesh,
        scratch_types=[
            pltpu.VMEM((CH,), jnp.int32),
            pltpu.VMEM((CH,), jnp.int32),
            pltpu.VMEM((CH, D), jnp.float32),
            pltpu.VMEM_SHARED((NP, D), jnp.float32),
            pltpu.SemaphoreType.DMA,
        ],
    )
    def _agg(y_hbm, src_hbm, dst_hbm, out_hbm, src_v, dst_v, rows_v, acc_sh,
             sem):
        c = lax.axis_index("c")
        s = lax.axis_index("s")
        wid = s * NC + c

        def zrow(i, carry):
            for j in range(D // 16):
                rows_v[i, pl.ds(j * 16, 16)] = jnp.zeros((16,), jnp.float32)
            return carry

        lax.fori_loop(0, CH, zrow, 0)
        for k in range(RPS // CH):
            pltpu.sync_copy(rows_v, acc_sh.at[pl.ds(s * RPS + k * CH, CH)])
        plsc.subcore_barrier()

        def body(it, carry):
            base = wid * EPT + it * CH
            pltpu.sync_copy(src_hbm.at[pl.ds(base, CH)], src_v)
            pltpu.async_copy(y_hbm.at[src_v], rows_v, sem).wait()
            pltpu.sync_copy(dst_hbm.at[pl.ds(base, CH)], dst_v)
            pltpu.sync_copy(rows_v, acc_sh.at[dst_v], add=True)
            return carry

        lax.fori_loop(0, ITERS, body, 0)
        plsc.subcore_barrier()
        pltpu.sync_copy(acc_sh.at[pl.ds(s * RPS, RPS)],
                        out_hbm.at[pl.ds(c * NP + s * RPS, RPS)])

    return _agg


BR = 1024


def _tc1_body(x_ref, w1_ref, d0_ref, d1_ref, xw_ref, y1_ref, dis_ref):
    xw = jnp.dot(x_ref[...], w1_ref[...], preferred_element_type=jnp.float32)
    deg = d0_ref[...] + d1_ref[...] + 1.0
    dis = lax.rsqrt(deg)
    xw_ref[...] = xw
    y1_ref[...] = xw * dis
    dis_ref[...] = dis


_tc1 = pl.pallas_call(
    _tc1_body,
    grid=(NP // BR,),
    in_specs=[
        pl.BlockSpec((BR, D_IN), lambda i: (i, 0)),
        pl.BlockSpec((D_IN, HIDDEN), lambda i: (0, 0)),
        pl.BlockSpec((BR, 1), lambda i: (i, 0)),
        pl.BlockSpec((BR, 1), lambda i: (i, 0)),
    ],
    out_specs=[
        pl.BlockSpec((BR, HIDDEN), lambda i: (i, 0)),
        pl.BlockSpec((BR, HIDDEN), lambda i: (i, 0)),
        pl.BlockSpec((BR, 1), lambda i: (i, 0)),
    ],
    out_shape=[
        jax.ShapeDtypeStruct((NP, HIDDEN), jnp.float32),
        jax.ShapeDtypeStruct((NP, HIDDEN), jnp.float32),
        jax.ShapeDtypeStruct((NP, 1), jnp.float32),
    ],
)


def _tc2_body(a0_ref, a1_ref, xw_ref, dis_ref, b1_ref, w2_ref, hw_ref,
              y2_ref):
    dis = dis_ref[...]
    h = (a0_ref[...] + a1_ref[...]) * dis + xw_ref[...] * (dis * dis)
    h = jnp.maximum(h + b1_ref[...], 0.0)
    hw = jnp.dot(h, w2_ref[...], preferred_element_type=jnp.float32)
    hw_ref[...] = hw
    y2_ref[...] = hw * dis


_tc2 = pl.pallas_call(
    _tc2_body,
    grid=(NP // BR,),
    in_specs=[
        pl.BlockSpec((BR, HIDDEN), lambda i: (i, 0)),
        pl.BlockSpec((BR, HIDDEN), lambda i: (i, 0)),
        pl.BlockSpec((BR, HIDDEN), lambda i: (i, 0)),
        pl.BlockSpec((BR, 1), lambda i: (i, 0)),
        pl.BlockSpec((1, HIDDEN), lambda i: (0, 0)),
        pl.BlockSpec((HIDDEN, D2P), lambda i: (0, 0)),
    ],
    out_specs=[
        pl.BlockSpec((BR, D2P), lambda i: (i, 0)),
        pl.BlockSpec((BR, D2P), lambda i: (i, 0)),
    ],
    out_shape=[
        jax.ShapeDtypeStruct((NP, D2P), jnp.float32),
        jax.ShapeDtypeStruct((NP, D2P), jnp.float32),
    ],
)


def _tc3_body(a0_ref, a1_ref, hw_ref, dis_ref, b2_ref, o_ref):
    dis = dis_ref[...]
    o_ref[...] = ((a0_ref[...] + a1_ref[...]) * dis
                  + hw_ref[...] * (dis * dis) + b2_ref[...])


_tc3 = pl.pallas_call(
    _tc3_body,
    grid=(NP // BR,),
    in_specs=[
        pl.BlockSpec((BR, D2P), lambda i: (i, 0)),
        pl.BlockSpec((BR, D2P), lambda i: (i, 0)),
        pl.BlockSpec((BR, D2P), lambda i: (i, 0)),
        pl.BlockSpec((BR, 1), lambda i: (i, 0)),
        pl.BlockSpec((1, D2P), lambda i: (0, 0)),
    ],
    out_specs=pl.BlockSpec((BR, D2P), lambda i: (i, 0)),
    out_shape=jax.ShapeDtypeStruct((NP, D2P), jnp.float32),
)


def kernel(x, edge_index, W1, b1, W2, b2):
    src = edge_index[0]
    dst = edge_index[1]
    x_p = jnp.zeros((NP, D_IN), x.dtype).at[:N].set(x)
    W2_p = jnp.zeros((HIDDEN, D2P), W2.dtype).at[:, :CLS].set(W2)
    b2_p = jnp.zeros((1, D2P), b2.dtype).at[0, :CLS].set(b2)

    degp = _get_sc_degree()(dst)
    d0 = degp[:NP].reshape(NP, 1)
    d1 = degp[NP:].reshape(NP, 1)

    xw, y1, dis = _tc1(x_p, W1, d0, d1)
    acc1 = _make_sc_agg(HIDDEN)(y1, src, dst)
    hw, y2 = _tc2(acc1[:NP], acc1[NP:], xw, dis, b1.reshape(1, HIDDEN), W2_p)
    acc2 = _make_sc_agg(D2P)(y2, src, dst)
    out_p = _tc3(acc2[:NP], acc2[NP:], hw, dis, b2_p)
    return out_p[:N, :CLS]
